# baseline (speedup 1.0000x reference)
import functools

import jax
import jax.numpy as jnp
from jax import lax
from jax.experimental import pallas as pl
from jax.experimental.pallas import tpu as pltpu
from jax.experimental.pallas import tpu_sc as plsc

_NC = 2
_NS = 16
_NW = _NC * _NS
_LANES = 16
_CHUNK = 80
_NBUF = 6
_PF = 3


def _sc_mesh():
    return plsc.VectorSubcoreMesh(core_axis_name="c", subcore_axis_name="s")


def _make_degree_kernel(n, e):
    epw = e // _NW
    epc = epw // _CHUNK
    nz = n // _LANES

    @functools.partial(
        pl.kernel,
        out_type=[jax.ShapeDtypeStruct((_NW * n,), jnp.float32),
                  jax.ShapeDtypeStruct((_NW * n,), jnp.float32)],
        mesh=_sc_mesh(),
        compiler_params=pltpu.CompilerParams(
            needs_layout_passes=False, use_tc_tiling_on_sc=False),
        scratch_types=[
            pltpu.VMEM((epc, _CHUNK), jnp.int32),
            pltpu.VMEM((epc, _CHUNK), jnp.int32),
            pltpu.VMEM((n,), jnp.float32),
            pltpu.VMEM((n,), jnp.float32),
        ],
    )
    def deg_kernel(src_hbm, dst_hbm, osrc_hbm, odst_hbm,
                   sidx_v, didx_v, dsrc_v, ddst_v):
        cid = lax.axis_index("c")
        sid = lax.axis_index("s")
        wid = sid * _NC + cid
        pltpu.sync_copy(src_hbm.at[wid], sidx_v)
        pltpu.sync_copy(dst_hbm.at[wid], didx_v)
        zero16 = jnp.zeros((_LANES,), jnp.float32)

        def zb(i, carry):
            dsrc_v[pl.ds(i * _LANES, _LANES)] = zero16
            ddst_v[pl.ds(i * _LANES, _LANES)] = zero16
            return carry

        lax.fori_loop(0, nz, zb, 0)

        one16 = jnp.ones((_LANES,), jnp.float32)
        vecs_per_chunk = _CHUNK // _LANES

        def body(k, carry):
            def inner(j, c2):
                si = sidx_v[k, pl.ds(j * _LANES, _LANES)]
                plsc.addupdate_scatter(dsrc_v, [si], one16)
                di = didx_v[k, pl.ds(j * _LANES, _LANES)]
                plsc.addupdate_scatter(ddst_v, [di], one16)
                return c2

            return lax.fori_loop(0, vecs_per_chunk, inner, carry)

        lax.fori_loop(0, epc, body, 0)

        pltpu.sync_copy(dsrc_v, osrc_hbm.at[pl.ds(wid * n, n)])
        pltpu.sync_copy(ddst_v, odst_hbm.at[pl.ds(wid * n, n)])

    return deg_kernel


def _make_agg_kernel(n, e, d, fused):
    epw = e // _NW
    epc = epw // _CHUNK
    npt = n // _NS

    @functools.partial(
        pl.kernel,
        out_type=jax.ShapeDtypeStruct((_NC * n, d), jnp.float32),
        mesh=_sc_mesh(),
        compiler_params=pltpu.CompilerParams(
            needs_layout_passes=False, use_tc_tiling_on_sc=False),
        scratch_types=[
            pltpu.VMEM((epc, _CHUNK), jnp.int32),
            pltpu.VMEM((epc, _CHUNK), jnp.int32),
            pltpu.VMEM((_NBUF, _CHUNK, d), jnp.float32),
            pltpu.VMEM((npt, d), jnp.float32),
            pltpu.VMEM((npt, d), jnp.float32),
            pltpu.VMEM((npt, 3 * d), jnp.float32),
            pltpu.VMEM((d,), jnp.float32),
            pltpu.VMEM_SHARED((n, d), jnp.float32),
            pltpu.VMEM_SHARED((n, d), jnp.float32),
            pltpu.SemaphoreType.DMA((_NBUF,)),
            pltpu.SemaphoreType.DMA((_NBUF,)),
        ],
    )
    def agg_kernel(*refs):
        if fused:
            (p_hbm, htab_hbm, b1_hbm, src_hbm, dst_hbm, out_hbm,
             sidx_v, didx_v, rows_v, zrow_v, pbuf_v, sbuf_v, b1_v,
             tab_sh, acc_sh, sem_g, sem_s) = refs
        else:
            (htab_hbm, src_hbm, dst_hbm, out_hbm,
             sidx_v, didx_v, rows_v, zrow_v, pbuf_v, sbuf_v, b1_v,
             tab_sh, acc_sh, sem_g, sem_s) = refs
        cid = lax.axis_index("c")
        sid = lax.axis_index("s")
        wid = sid * _NC + cid
        pltpu.sync_copy(src_hbm.at[wid], sidx_v)
        pltpu.sync_copy(dst_hbm.at[wid], didx_v)
        row_slice = pl.ds(sid * npt, npt)
        pltpu.sync_copy(htab_hbm.at[row_slice], sbuf_v)
        if not fused:
            def hrow(i, carry):
                zrow_v[i] = sbuf_v[i, pl.ds(0, d)]
                return carry

            lax.fori_loop(0, npt, hrow, 0)
        else:
            pltpu.sync_copy(p_hbm.at[pl.ds(sid * npt, npt)], zrow_v)
            pltpu.sync_copy(p_hbm.at[pl.ds(n + sid * npt, npt)], pbuf_v)
            pltpu.sync_copy(b1_hbm, b1_v)
            b1r = b1_v[...]

            def trow(i, carry):
                row = ((zrow_v[i] + pbuf_v[i]) * sbuf_v[i, pl.ds(d, d)]
                       + b1r)
                zrow_v[i] = jnp.maximum(row, 0.0) * sbuf_v[i, pl.ds(2 * d, d)]
                return carry

            lax.fori_loop(0, npt, trow, 0)
        pltpu.sync_copy(zrow_v, tab_sh.at[row_slice])
        zero16 = jnp.zeros((_LANES,), jnp.float32)

        def zb(i, carry):
            zrow_v[i] = zero16
            return carry

        lax.fori_loop(0, npt, zb, 0)
        pltpu.sync_copy(zrow_v, acc_sh.at[row_slice])
        plsc.subcore_barrier()

        for j in range(_PF):
            pltpu.async_copy(tab_sh.at[sidx_v.at[j]], rows_v.at[j],
                             sem_g.at[j])
        for k in range(_NBUF - _PF):
            kp = k + _PF
            pltpu.async_copy(tab_sh.at[sidx_v.at[kp]], rows_v.at[kp % _NBUF],
                             sem_g.at[kp % _NBUF])
            pltpu.make_async_copy(tab_sh.at[sidx_v.at[k]],
                                  rows_v.at[k % _NBUF],
                                  sem_g.at[k % _NBUF]).wait()
            pltpu.async_copy(rows_v.at[k % _NBUF], acc_sh.at[didx_v.at[k]],
                             sem_s.at[k % _NBUF], add=True)

        def body(k, carry):
            kp = k + _PF
            bp = lax.rem(kp, _NBUF)
            pltpu.make_async_copy(rows_v.at[bp],
                                  acc_sh.at[didx_v.at[kp - _NBUF]],
                                  sem_s.at[bp]).wait()
            pltpu.async_copy(tab_sh.at[sidx_v.at[kp]], rows_v.at[bp],
                             sem_g.at[bp])
            b = lax.rem(k, _NBUF)
            pltpu.make_async_copy(tab_sh.at[sidx_v.at[k]], rows_v.at[b],
                                  sem_g.at[b]).wait()
            pltpu.async_copy(rows_v.at[b], acc_sh.at[didx_v.at[k]],
                             sem_s.at[b], add=True)
            return carry

        lax.fori_loop(_NBUF - _PF, epc - _PF, body, 0)
        for k in range(epc - _PF, epc):
            pltpu.make_async_copy(tab_sh.at[sidx_v.at[k]],
                                  rows_v.at[k % _NBUF],
                                  sem_g.at[k % _NBUF]).wait()
            pltpu.async_copy(rows_v.at[k % _NBUF], acc_sh.at[didx_v.at[k]],
                             sem_s.at[k % _NBUF], add=True)
        for m in range(epc - _NBUF, epc):
            pltpu.make_async_copy(rows_v.at[m % _NBUF],
                                  acc_sh.at[didx_v.at[m]],
                                  sem_s.at[m % _NBUF]).wait()

        plsc.subcore_barrier()

        if fused:
            pltpu.sync_copy(acc_sh.at[row_slice], zrow_v)

            def scale(i, carry):
                zrow_v[i] = zrow_v[i] * sbuf_v[i, pl.ds(d, d)]
                return carry

            lax.fori_loop(0, npt, scale, 0)
            pltpu.sync_copy(zrow_v,
                            out_hbm.at[pl.ds(cid * n + sid * npt, npt)])
        else:
            @pl.when(sid == 0)
            def _out():
                pltpu.sync_copy(acc_sh, out_hbm.at[pl.ds(cid * n, n)])

    return agg_kernel


def _dense1(x, w1, psrc, pdst):
    n, _ = x.shape
    dh = w1.shape[1]

    def body(x_ref, w1_ref, ps_ref, pd_ref, o_ref):
        dout = jnp.maximum(jnp.sum(ps_ref[...], axis=1, keepdims=True), 1.0)
        dinn = jnp.maximum(jnp.sum(pd_ref[...], axis=1, keepdims=True), 1.0)
        dor = lax.rsqrt(dout)
        dirc = lax.rsqrt(dinn)
        h0 = jnp.dot(x_ref[...] * dor, w1_ref[...],
                     preferred_element_type=jnp.float32)
        o_ref[...] = jnp.concatenate(
            [h0, jnp.broadcast_to(dirc, (n, dh)),
             jnp.broadcast_to(dor, (n, dh))], axis=1)

    return pl.pallas_call(
        body,
        out_shape=jax.ShapeDtypeStruct((n, 3 * dh), jnp.float32),
    )(x, w1, psrc, pdst)


def _dense3(q, w2, b2):
    _, n, _ = q.shape
    do = w2.shape[1]

    def body(q_ref, w2_ref, b2_ref, o_ref):
        o_ref[...] = jnp.dot(q_ref[0] + q_ref[1], w2_ref[...],
                             preferred_element_type=jnp.float32) + b2_ref[...]

    return pl.pallas_call(
        body,
        out_shape=jax.ShapeDtypeStruct((n, do), jnp.float32),
    )(q, w2, b2)


def kernel(node_feat, edge_index, edge_feat, W1, b1, W2, b2):
    n, _ = node_feat.shape
    e = edge_index.shape[1]
    d_hid = W1.shape[1]
    epc = e // _NW // _CHUNK

    src3 = edge_index[0].reshape(_NW, epc, _CHUNK)
    dst3 = edge_index[1].reshape(_NW, epc, _CHUNK)

    psrc, pdst = _make_degree_kernel(n, e)(src3, dst3)
    htab = _dense1(node_feat, W1,
                   psrc.reshape(_NW, n).T, pdst.reshape(_NW, n).T)
    p1 = _make_agg_kernel(n, e, d_hid, fused=False)(htab, src3, dst3)
    q = _make_agg_kernel(n, e, d_hid, fused=True)(
        p1, htab, b1, src3, dst3)
    return _dense3(q.reshape(_NC, n, d_hid), W2, b2)


# speedup vs baseline: 1.0328x; 1.0328x over previous
import functools

import jax
import jax.numpy as jnp
from jax import lax
from jax.experimental import pallas as pl
from jax.experimental.pallas import tpu as pltpu
from jax.experimental.pallas import tpu_sc as plsc

_NC = 2
_NS = 16
_NW = _NC * _NS
_LANES = 16
_CHUNK = 80
_NBUF = 6
_PF = 3


def _sc_mesh():
    return plsc.VectorSubcoreMesh(core_axis_name="c", subcore_axis_name="s")


def _make_degree_kernel(n, e):
    epw = e // _NW
    epc = epw // _CHUNK
    nz = n // _LANES

    @functools.partial(
        pl.kernel,
        out_type=[jax.ShapeDtypeStruct((_NW * n,), jnp.float32),
                  jax.ShapeDtypeStruct((_NW * n,), jnp.float32)],
        mesh=_sc_mesh(),
        compiler_params=pltpu.CompilerParams(
            needs_layout_passes=False, use_tc_tiling_on_sc=False),
        scratch_types=[
            pltpu.VMEM((epc, _CHUNK), jnp.int32),
            pltpu.VMEM((epc, _CHUNK), jnp.int32),
            pltpu.VMEM((n,), jnp.float32),
            pltpu.VMEM((n,), jnp.float32),
        ],
    )
    def deg_kernel(src_hbm, dst_hbm, osrc_hbm, odst_hbm,
                   sidx_v, didx_v, dsrc_v, ddst_v):
        cid = lax.axis_index("c")
        sid = lax.axis_index("s")
        wid = sid * _NC + cid
        pltpu.sync_copy(src_hbm.at[wid], sidx_v)
        pltpu.sync_copy(dst_hbm.at[wid], didx_v)
        zero16 = jnp.zeros((_LANES,), jnp.float32)

        def zb(i, carry):
            dsrc_v[pl.ds(i * _LANES, _LANES)] = zero16
            ddst_v[pl.ds(i * _LANES, _LANES)] = zero16
            return carry

        lax.fori_loop(0, nz, zb, 0)

        one16 = jnp.ones((_LANES,), jnp.float32)
        vecs_per_chunk = _CHUNK // _LANES

        def body(k, carry):
            def inner(j, c2):
                si = sidx_v[k, pl.ds(j * _LANES, _LANES)]
                plsc.addupdate_scatter(dsrc_v, [si], one16)
                di = didx_v[k, pl.ds(j * _LANES, _LANES)]
                plsc.addupdate_scatter(ddst_v, [di], one16)
                return c2

            return lax.fori_loop(0, vecs_per_chunk, inner, carry)

        lax.fori_loop(0, epc, body, 0)

        pltpu.sync_copy(dsrc_v, osrc_hbm.at[pl.ds(wid * n, n)])
        pltpu.sync_copy(ddst_v, odst_hbm.at[pl.ds(wid * n, n)])

    return deg_kernel


def _make_agg_kernel(n, e, d, fused):
    epw = e // _NW
    epc = epw // _CHUNK
    npt = n // _NS

    @functools.partial(
        pl.kernel,
        out_type=jax.ShapeDtypeStruct((_NC * n, d), jnp.float32),
        mesh=_sc_mesh(),
        compiler_params=pltpu.CompilerParams(
            needs_layout_passes=False, use_tc_tiling_on_sc=False),
        scratch_types=[
            pltpu.VMEM((epc, _CHUNK), jnp.int32),
            pltpu.VMEM((epc, _CHUNK), jnp.int32),
            pltpu.VMEM((_NBUF, _CHUNK, d), jnp.float32),
            pltpu.VMEM((npt, d), jnp.float32),
            pltpu.VMEM((npt, d), jnp.float32),
            pltpu.VMEM((npt, 2 * d), jnp.float32),
            pltpu.VMEM((d,), jnp.float32),
            pltpu.VMEM_SHARED((n, d), jnp.float32),
            pltpu.VMEM_SHARED((n, d), jnp.float32),
            pltpu.SemaphoreType.DMA((_NBUF,)),
            pltpu.SemaphoreType.DMA((_NBUF,)),
        ],
    )
    def agg_kernel(*refs):
        if fused:
            (p_hbm, sc_hbm, b1_hbm, src_hbm, dst_hbm, out_hbm,
             sidx_v, didx_v, rows_v, zrow_v, pbuf_v, sbuf_v, b1_v,
             tab_sh, acc_sh, sem_g, sem_s) = refs
        else:
            (tab_hbm, src_hbm, dst_hbm, out_hbm,
             sidx_v, didx_v, rows_v, zrow_v, pbuf_v, sbuf_v, b1_v,
             tab_sh, acc_sh, sem_g, sem_s) = refs
        cid = lax.axis_index("c")
        sid = lax.axis_index("s")
        wid = sid * _NC + cid
        pltpu.sync_copy(src_hbm.at[wid], sidx_v)
        pltpu.sync_copy(dst_hbm.at[wid], didx_v)
        row_slice = pl.ds(sid * npt, npt)
        if not fused:
            pltpu.sync_copy(tab_hbm.at[row_slice], zrow_v)
        else:
            pltpu.sync_copy(sc_hbm.at[row_slice], sbuf_v)
            pltpu.sync_copy(p_hbm.at[pl.ds(sid * npt, npt)], zrow_v)
            pltpu.sync_copy(p_hbm.at[pl.ds(n + sid * npt, npt)], pbuf_v)
            pltpu.sync_copy(b1_hbm, b1_v)
            b1r = b1_v[...]

            def trow(i, carry):
                row = ((zrow_v[i] + pbuf_v[i]) * sbuf_v[i, pl.ds(0, d)]
                       + b1r)
                zrow_v[i] = jnp.maximum(row, 0.0) * sbuf_v[i, pl.ds(d, d)]
                return carry

            lax.fori_loop(0, npt, trow, 0)
        pltpu.sync_copy(zrow_v, tab_sh.at[row_slice])
        zero16 = jnp.zeros((_LANES,), jnp.float32)

        def zb(i, carry):
            zrow_v[i] = zero16
            return carry

        lax.fori_loop(0, npt, zb, 0)
        pltpu.sync_copy(zrow_v, acc_sh.at[row_slice])
        plsc.subcore_barrier()

        for j in range(_PF):
            pltpu.async_copy(tab_sh.at[sidx_v.at[j]], rows_v.at[j],
                             sem_g.at[j])
        for k in range(_NBUF - _PF):
            kp = k + _PF
            pltpu.async_copy(tab_sh.at[sidx_v.at[kp]], rows_v.at[kp % _NBUF],
                             sem_g.at[kp % _NBUF])
            pltpu.make_async_copy(tab_sh.at[sidx_v.at[k]],
                                  rows_v.at[k % _NBUF],
                                  sem_g.at[k % _NBUF]).wait()
            pltpu.async_copy(rows_v.at[k % _NBUF], acc_sh.at[didx_v.at[k]],
                             sem_s.at[k % _NBUF], add=True)

        def body(k, carry):
            kp = k + _PF
            bp = lax.rem(kp, _NBUF)
            pltpu.make_async_copy(rows_v.at[bp],
                                  acc_sh.at[didx_v.at[kp - _NBUF]],
                                  sem_s.at[bp]).wait()
            pltpu.async_copy(tab_sh.at[sidx_v.at[kp]], rows_v.at[bp],
                             sem_g.at[bp])
            b = lax.rem(k, _NBUF)
            pltpu.make_async_copy(tab_sh.at[sidx_v.at[k]], rows_v.at[b],
                                  sem_g.at[b]).wait()
            pltpu.async_copy(rows_v.at[b], acc_sh.at[didx_v.at[k]],
                             sem_s.at[b], add=True)
            return carry

        lax.fori_loop(_NBUF - _PF, epc - _PF, body, 0)
        for k in range(epc - _PF, epc):
            pltpu.make_async_copy(tab_sh.at[sidx_v.at[k]],
                                  rows_v.at[k % _NBUF],
                                  sem_g.at[k % _NBUF]).wait()
            pltpu.async_copy(rows_v.at[k % _NBUF], acc_sh.at[didx_v.at[k]],
                             sem_s.at[k % _NBUF], add=True)
        for m in range(epc - _NBUF, epc):
            pltpu.make_async_copy(rows_v.at[m % _NBUF],
                                  acc_sh.at[didx_v.at[m]],
                                  sem_s.at[m % _NBUF]).wait()

        plsc.subcore_barrier()

        if fused:
            pltpu.sync_copy(acc_sh.at[row_slice], zrow_v)

            def scale(i, carry):
                zrow_v[i] = zrow_v[i] * sbuf_v[i, pl.ds(0, d)]
                return carry

            lax.fori_loop(0, npt, scale, 0)
            pltpu.sync_copy(zrow_v,
                            out_hbm.at[pl.ds(cid * n + sid * npt, npt)])
        else:
            @pl.when(sid == 0)
            def _out():
                pltpu.sync_copy(acc_sh, out_hbm.at[pl.ds(cid * n, n)])

    return agg_kernel


def _dense1(x, w1, psrc, pdst):
    n, _ = x.shape
    dh = w1.shape[1]

    def body(x_ref, w1_ref, ps_ref, pd_ref, o_ref, s_ref):
        dout = jnp.maximum(jnp.sum(ps_ref[...], axis=1, keepdims=True), 1.0)
        dinn = jnp.maximum(jnp.sum(pd_ref[...], axis=1, keepdims=True), 1.0)
        dor = lax.rsqrt(dout)
        dirc = lax.rsqrt(dinn)
        h0 = jnp.dot(x_ref[...] * dor, w1_ref[...],
                     preferred_element_type=jnp.float32)
        o_ref[...] = h0
        s_ref[...] = jnp.concatenate(
            [jnp.broadcast_to(dirc, (n, dh)),
             jnp.broadcast_to(dor, (n, dh))], axis=1)

    return pl.pallas_call(
        body,
        out_shape=[jax.ShapeDtypeStruct((n, dh), jnp.float32),
                   jax.ShapeDtypeStruct((n, 2 * dh), jnp.float32)],
    )(x, w1, psrc, pdst)


def _dense3(q, w2, b2):
    _, n, _ = q.shape
    do = w2.shape[1]

    def body(q_ref, w2_ref, b2_ref, o_ref):
        o_ref[...] = jnp.dot(q_ref[0] + q_ref[1], w2_ref[...],
                             preferred_element_type=jnp.float32) + b2_ref[...]

    return pl.pallas_call(
        body,
        out_shape=jax.ShapeDtypeStruct((n, do), jnp.float32),
    )(q, w2, b2)


def kernel(node_feat, edge_index, edge_feat, W1, b1, W2, b2):
    n, _ = node_feat.shape
    e = edge_index.shape[1]
    d_hid = W1.shape[1]
    epc = e // _NW // _CHUNK

    src3 = edge_index[0].reshape(_NW, epc, _CHUNK)
    dst3 = edge_index[1].reshape(_NW, epc, _CHUNK)

    psrc, pdst = _make_degree_kernel(n, e)(src3, dst3)
    h0, scl = _dense1(node_feat, W1,
                      psrc.reshape(_NW, n).T, pdst.reshape(_NW, n).T)
    p1 = _make_agg_kernel(n, e, d_hid, fused=False)(h0, src3, dst3)
    q = _make_agg_kernel(n, e, d_hid, fused=True)(
        p1, scl, b1, src3, dst3)
    return _dense3(q.reshape(_NC, n, d_hid), W2, b2)
